# Initial kernel scaffold; baseline (speedup 1.0000x reference)
import jax
import jax.numpy as jnp
from jax.experimental import pallas as pl


def kernel(input_ids, category_ids, id_table, cat_table, W, b, gamma, beta):
    raise NotImplementedError("write your pallas kernel here")


# speedup vs baseline: 1.8662x; 1.8662x over previous
import functools

import jax
import jax.numpy as jnp
from jax import lax
from jax.experimental import pallas as pl
from jax.experimental.pallas import tpu as pltpu
from jax.experimental.pallas import tpu_sc as plsc

EMB = 64
HID = 128
EPS = 1e-12

_NC = 2
_NS = 16
_NW = _NC * _NS

_IDXW = 128
_K = 4


def _sc_gather_pair(ids2d, cats2d, id_table, cat_table):
  NR = ids2d.shape[0]
  rpw = NR // _NW

  mesh = plsc.VectorSubcoreMesh(core_axis_name="c", subcore_axis_name="s")

  @functools.partial(
      pl.kernel,
      mesh=mesh,
      compiler_params=pltpu.CompilerParams(use_tc_tiling_on_sc=False),
      out_type=[
          jax.ShapeDtypeStruct((NR, _IDXW, EMB), jnp.float32),
          jax.ShapeDtypeStruct((NR, _IDXW, EMB), jnp.float32),
      ],
      scratch_types=[
          pltpu.VMEM((_K, _IDXW), jnp.int32),
          pltpu.VMEM((_K, _IDXW), jnp.int32),
          pltpu.VMEM((_K, _IDXW, EMB), jnp.float32),
          pltpu.VMEM((_K, _IDXW, EMB), jnp.float32),
          pltpu.SemaphoreType.DMA,
          pltpu.SemaphoreType.DMA,
      ],
  )
  def k(ids_hbm, cats_hbm, idt_hbm, catt_hbm, out_id, out_cat,
        idx_i, idx_c, rows_i, rows_c, sem_i, sem_c):
    wid = lax.axis_index("s") * _NC + lax.axis_index("c")
    base = wid * rpw

    def chunk(g, carry):
      rb = base + g * _K
      pltpu.sync_copy(ids_hbm.at[pl.ds(rb, _K)], idx_i)
      pltpu.sync_copy(cats_hbm.at[pl.ds(rb, _K)], idx_c)
      cps = []
      for j in range(_K):
        cps.append(pltpu.async_copy(idt_hbm.at[idx_i.at[j]], rows_i.at[j],
                                    sem_i))
        cps.append(pltpu.async_copy(catt_hbm.at[idx_c.at[j]], rows_c.at[j],
                                    sem_c))
      for c in cps:
        c.wait()
      pltpu.sync_copy(rows_i, out_id.at[pl.ds(rb, _K)])
      pltpu.sync_copy(rows_c, out_cat.at[pl.ds(rb, _K)])
      return carry

    lax.fori_loop(0, rpw // _K, chunk, 0)

  return k(ids2d, cats2d, id_table, cat_table)


def _tc_fused(eid, ecat, w_id_t, w_cat_t, b, gamma, beta):
  N = eid.shape[0]
  T = 2048
  grid = (N // T,)

  def body(eid_ref, ecat_ref, wi_ref, wc_ref, b_ref, g_ref, bt_ref, o_ref):
    h = jnp.dot(eid_ref[...], wi_ref[...], preferred_element_type=jnp.float32)
    h += jnp.dot(ecat_ref[...], wc_ref[...], preferred_element_type=jnp.float32)
    h += b_ref[...]
    mu = jnp.mean(h, axis=-1, keepdims=True)
    d = h - mu
    var = jnp.mean(d * d, axis=-1, keepdims=True)
    o_ref[...] = d * lax.rsqrt(var + EPS) * g_ref[...] + bt_ref[...]

  return pl.pallas_call(
      body,
      grid=grid,
      in_specs=[
          pl.BlockSpec((T, EMB), lambda i: (i, 0)),
          pl.BlockSpec((T, EMB), lambda i: (i, 0)),
          pl.BlockSpec((EMB, HID), lambda i: (0, 0)),
          pl.BlockSpec((EMB, HID), lambda i: (0, 0)),
          pl.BlockSpec((1, HID), lambda i: (0, 0)),
          pl.BlockSpec((1, HID), lambda i: (0, 0)),
          pl.BlockSpec((1, HID), lambda i: (0, 0)),
      ],
      out_specs=pl.BlockSpec((T, HID), lambda i: (i, 0)),
      out_shape=jax.ShapeDtypeStruct((N, HID), jnp.float32),
  )(eid, ecat, w_id_t, w_cat_t, b, gamma, beta)


def kernel(input_ids, category_ids, id_table, cat_table, W, b, gamma, beta):
  B, L = input_ids.shape
  N = B * L
  NR = N // _IDXW

  ids2d = input_ids.reshape(NR, _IDXW).astype(jnp.int32)
  cats2d = category_ids.reshape(NR, _IDXW).astype(jnp.int32)

  eid, ecat = _sc_gather_pair(ids2d, cats2d, id_table, cat_table)

  w_id_t = W[:, :EMB].T
  w_cat_t = W[:, EMB:].T
  out = _tc_fused(
      eid.reshape(N, EMB), ecat.reshape(N, EMB), w_id_t, w_cat_t,
      b.reshape(1, HID), gamma.reshape(1, HID), beta.reshape(1, HID))
  return out.reshape(B, L, HID)


# speedup vs baseline: 2.7605x; 1.4792x over previous
import functools

import jax
import jax.numpy as jnp
from jax import lax
from jax.experimental import pallas as pl
from jax.experimental.pallas import tpu as pltpu
from jax.experimental.pallas import tpu_sc as plsc

EMB = 64
HID = 128
EPS = 1e-12

_NC = 2
_NS = 16
_NW = _NC * _NS

_IDXW = 128
_K = 4


def _sc_gather_concat(ids2d, cats2d, id_table, cat_table):
  NR = ids2d.shape[0]
  rpw = NR // _NW

  mesh = plsc.VectorSubcoreMesh(core_axis_name="c", subcore_axis_name="s")

  @functools.partial(
      pl.kernel,
      mesh=mesh,
      compiler_params=pltpu.CompilerParams(use_tc_tiling_on_sc=False),
      out_type=jax.ShapeDtypeStruct((NR, _IDXW, 2 * EMB), jnp.float32),
      scratch_types=[
          pltpu.VMEM((_K, _IDXW), jnp.int32),
          pltpu.VMEM((_K, _IDXW), jnp.int32),
          pltpu.VMEM((_K, _IDXW, EMB), jnp.float32),
          pltpu.VMEM((_K, _IDXW, EMB), jnp.float32),
          pltpu.SemaphoreType.DMA,
          pltpu.SemaphoreType.DMA,
      ],
  )
  def k(ids_hbm, cats_hbm, idt_hbm, catt_hbm, out,
        idx_i, idx_c, rows_i, rows_c, sem_i, sem_c):
    wid = lax.axis_index("s") * _NC + lax.axis_index("c")
    base = wid * rpw

    def chunk(g, carry):
      rb = base + g * _K
      pltpu.sync_copy(ids_hbm.at[pl.ds(rb, _K)], idx_i)
      pltpu.sync_copy(cats_hbm.at[pl.ds(rb, _K)], idx_c)
      cps = []
      for j in range(_K):
        cps.append(pltpu.async_copy(
            idt_hbm.at[idx_i.at[j]], rows_i.at[j], sem_i))
        cps.append(pltpu.async_copy(
            catt_hbm.at[idx_c.at[j]], rows_c.at[j], sem_c))
      for c in cps:
        c.wait()
      pltpu.sync_copy(rows_i, out.at[pl.ds(rb, _K), :, pl.ds(0, EMB)])
      pltpu.sync_copy(rows_c, out.at[pl.ds(rb, _K), :, pl.ds(EMB, EMB)])
      return carry

    lax.fori_loop(0, rpw // _K, chunk, 0)

  return k(ids2d, cats2d, id_table, cat_table)


def _tc_fused(emb, w_t, b, gamma, beta):
  N = emb.shape[0]
  T = 2048
  grid = (N // T,)

  def body(e_ref, w_ref, b_ref, g_ref, bt_ref, o_ref):
    h = jnp.dot(e_ref[...], w_ref[...], preferred_element_type=jnp.float32)
    h += b_ref[...]
    mu = jnp.mean(h, axis=-1, keepdims=True)
    d = h - mu
    var = jnp.mean(d * d, axis=-1, keepdims=True)
    o_ref[...] = d * lax.rsqrt(var + EPS) * g_ref[...] + bt_ref[...]

  return pl.pallas_call(
      body,
      grid=grid,
      in_specs=[
          pl.BlockSpec((T, 2 * EMB), lambda i: (i, 0)),
          pl.BlockSpec((2 * EMB, HID), lambda i: (0, 0)),
          pl.BlockSpec((1, HID), lambda i: (0, 0)),
          pl.BlockSpec((1, HID), lambda i: (0, 0)),
          pl.BlockSpec((1, HID), lambda i: (0, 0)),
      ],
      out_specs=pl.BlockSpec((T, HID), lambda i: (i, 0)),
      out_shape=jax.ShapeDtypeStruct((N, HID), jnp.float32),
  )(emb, w_t, b, gamma, beta)


def kernel(input_ids, category_ids, id_table, cat_table, W, b, gamma, beta):
  B, L = input_ids.shape
  N = B * L
  NR = N // _IDXW

  ids2d = input_ids.reshape(NR, _IDXW).astype(jnp.int32)
  cats2d = category_ids.reshape(NR, _IDXW).astype(jnp.int32)

  emb = _sc_gather_concat(ids2d, cats2d, id_table, cat_table)

  out = _tc_fused(
      emb.reshape(N, 2 * EMB), W.T,
      b.reshape(1, HID), gamma.reshape(1, HID), beta.reshape(1, HID))
  return out.reshape(B, L, HID)
